# baseline (speedup 1.0000x reference)
import functools

import jax
import jax.numpy as jnp
from jax import lax
from jax.experimental import pallas as pl
from jax.experimental.pallas import tpu as pltpu
from jax.experimental.pallas import tpu_sc as plsc

BATCH = 16
LENGTH = 512
N_CHANNELS = 6
N_REL = 131072
TOTAL = BATCH * LENGTH * LENGTH * N_CHANNELS

ZROWS = BATCH * LENGTH
ZCOLS = LENGTH * N_CHANNELS
ZBLK = 512

ICOLS = 128
IROWS = N_REL // ICOLS
IBLK = IROWS // (ZROWS // ZBLK)

NUM_CORES = 2
NUM_SUBCORES = 16
NTILES = NUM_CORES * NUM_SUBCORES
ROWS_PER_TILE = IROWS // NTILES
DMA_GROUP = 8


def _tc_body(b_ref, s_ref, d_ref, t_ref, zero_ref, off_ref):
    zero_ref[...] = jnp.zeros_like(zero_ref)
    off_ref[...] = (
        (b_ref[...] * LENGTH + s_ref[...]) * LENGTH + d_ref[...]
    ) * N_CHANNELS + t_ref[...]


def _tc_zero_and_offsets(rb, rs, rd, rt):
    grid = ZROWS // ZBLK
    return pl.pallas_call(
        _tc_body,
        grid=(grid,),
        in_specs=[pl.BlockSpec((IBLK, ICOLS), lambda i: (i, 0))] * 4,
        out_specs=[
            pl.BlockSpec((ZBLK, ZCOLS), lambda i: (i, 0)),
            pl.BlockSpec((IBLK, ICOLS), lambda i: (i, 0)),
        ],
        out_shape=[
            jax.ShapeDtypeStruct((ZROWS, ZCOLS), jnp.float32),
            jax.ShapeDtypeStruct((IROWS, ICOLS), jnp.int32),
        ],
    )(rb, rs, rd, rt)


_mesh = plsc.VectorSubcoreMesh(
    core_axis_name="c", subcore_axis_name="s",
    num_cores=NUM_CORES, num_subcores=NUM_SUBCORES,
)


@functools.partial(
    pl.kernel,
    mesh=_mesh,
    out_type=(),
    scratch_types=[
        pltpu.VMEM((ROWS_PER_TILE, ICOLS), jnp.int32),
        pltpu.VMEM((ROWS_PER_TILE, ICOLS), jnp.float32),
        pltpu.SemaphoreType.DMA,
    ],
)
def _sc_scatter(off_hbm, val_hbm, dep_hbm, idx_v, val_v, sem):
    c = lax.axis_index("c")
    s = lax.axis_index("s")
    wid = s * NUM_CORES + c
    base = wid * ROWS_PER_TILE
    pltpu.sync_copy(off_hbm.at[pl.ds(base, ROWS_PER_TILE)], idx_v)
    pltpu.sync_copy(val_hbm.at[pl.ds(base, ROWS_PER_TILE)], val_v)

    def group(g, carry):
        row0 = g * DMA_GROUP
        copies = [
            pltpu.async_copy(
                val_v.at[row0 + j], dep_hbm.at[idx_v.at[row0 + j]], sem
            )
            for j in range(DMA_GROUP)
        ]
        for cp in copies:
            cp.wait()
        return carry

    lax.fori_loop(0, ROWS_PER_TILE // DMA_GROUP, group, 0)


def kernel(rel_b, rel_src, rel_dst, rel_type, rel_val):
    rb = rel_b.reshape(IROWS, ICOLS)
    rs = rel_src.reshape(IROWS, ICOLS)
    rd = rel_dst.reshape(IROWS, ICOLS)
    rt = rel_type.reshape(IROWS, ICOLS)
    zeros2d, off2d = _tc_zero_and_offsets(rb, rs, rd, rt)
    dep_ref = jax.new_ref(zeros2d.reshape(TOTAL))
    _sc_scatter(off2d, rel_val.reshape(IROWS, ICOLS), dep_ref)
    return dep_ref[...].reshape(BATCH, LENGTH, LENGTH, N_CHANNELS)


# speedup vs baseline: 1.0249x; 1.0249x over previous
import functools

import jax
import jax.numpy as jnp
from jax import lax
from jax.experimental import pallas as pl
from jax.experimental.pallas import tpu as pltpu
from jax.experimental.pallas import tpu_sc as plsc

BATCH = 16
LENGTH = 512
N_CHANNELS = 6
N_REL = 131072
TOTAL = BATCH * LENGTH * LENGTH * N_CHANNELS

NUM_CORES = 2
NUM_SUBCORES = 16
HALF = TOTAL // NUM_CORES
SHARD = HALF // NUM_SUBCORES
ZCHUNK = 32768
NZDMA = SHARD // ZCHUNK

ICOLS = 128
REL_PER_TILE = N_REL // NUM_SUBCORES
IROWS = REL_PER_TILE // ICOLS
VSTEPS = REL_PER_TILE // 16
IGNORED = -1
DMA_GROUP = 8

_mesh = plsc.VectorSubcoreMesh(
    core_axis_name="c", subcore_axis_name="s",
    num_cores=NUM_CORES, num_subcores=NUM_SUBCORES,
)


@functools.partial(
    pl.kernel,
    mesh=_mesh,
    out_type=jax.ShapeDtypeStruct((TOTAL,), jnp.float32),
    scratch_types=[
        pltpu.VMEM((ZCHUNK,), jnp.float32),
        pltpu.VMEM((REL_PER_TILE,), jnp.int32),
        pltpu.VMEM((REL_PER_TILE,), jnp.int32),
        pltpu.VMEM((REL_PER_TILE,), jnp.int32),
        pltpu.VMEM((REL_PER_TILE,), jnp.int32),
        pltpu.VMEM((REL_PER_TILE,), jnp.float32),
        pltpu.VMEM((IROWS, ICOLS), jnp.int32),
        pltpu.SemaphoreType.DMA,
        pltpu.SemaphoreType.DMA,
    ],
)
def _sc_dep(rb_hbm, rs_hbm, rd_hbm, rt_hbm, rv_hbm, dep_hbm,
            zero_v, b_v, s_v, d_v, t_v, v_v, idx_v, zsem, sem):
    cid = lax.axis_index("c")
    sid = lax.axis_index("s")

    def fill_zero(i, carry):
        zero_v[pl.ds(i * 16, 16)] = jnp.zeros((16,), jnp.float32)
        return carry

    lax.fori_loop(0, ZCHUNK // 16, fill_zero, 0)

    shard_base = cid * HALF + sid * SHARD
    zero_copies = [
        pltpu.make_async_copy(
            zero_v, dep_hbm.at[pl.ds(shard_base + k * ZCHUNK, ZCHUNK)], zsem
        )
        for k in range(NZDMA)
    ]
    for cp in zero_copies:
        cp.start()

    rel_base = sid * REL_PER_TILE
    in_copies = [
        pltpu.async_copy(hbm.at[pl.ds(rel_base, REL_PER_TILE)], vmem, sem)
        for hbm, vmem in [(rb_hbm, b_v), (rs_hbm, s_v), (rd_hbm, d_v),
                          (rt_hbm, t_v), (rv_hbm, v_v)]
    ]
    for cp in in_copies:
        cp.wait()

    lo = cid * HALF
    hi = lo + HALF

    def compute(m, carry):
        p = m * 16
        b = b_v[pl.ds(p, 16)]
        s = s_v[pl.ds(p, 16)]
        d = d_v[pl.ds(p, 16)]
        t = t_v[pl.ds(p, 16)]
        off = ((b * LENGTH + s) * LENGTH + d) * N_CHANNELS + t
        owned = (off >= lo) & (off < hi)
        r = m // 8
        c16 = (m % 8) * 16
        idx_v[r, pl.ds(c16, 16)] = jnp.where(
            owned, off, jnp.full((16,), IGNORED, jnp.int32)
        )
        return carry

    lax.fori_loop(0, VSTEPS, compute, 0)

    for cp in zero_copies:
        cp.wait()
    plsc.subcore_barrier()

    def group(g, carry):
        row0 = g * DMA_GROUP
        copies = [
            pltpu.async_copy(
                v_v.at[pl.ds((row0 + j) * ICOLS, ICOLS)],
                dep_hbm.at[plsc.Indices(idx_v.at[row0 + j],
                                        ignored_value=IGNORED)],
                sem,
            )
            for j in range(DMA_GROUP)
        ]
        for cp in copies:
            cp.wait()
        return carry

    lax.fori_loop(0, IROWS // DMA_GROUP, group, 0)


def kernel(rel_b, rel_src, rel_dst, rel_type, rel_val):
    dep = _sc_dep(rel_b, rel_src, rel_dst, rel_type, rel_val)
    return dep.reshape(BATCH, LENGTH, LENGTH, N_CHANNELS)


# speedup vs baseline: 14.6252x; 14.2694x over previous
import functools

import jax
import jax.numpy as jnp
from jax import lax
from jax.experimental import pallas as pl
from jax.experimental.pallas import tpu as pltpu
from jax.experimental.pallas import tpu_sc as plsc

BATCH = 16
LENGTH = 512
N_CHANNELS = 6
N_REL = 131072
TOTAL = BATCH * LENGTH * LENGTH * N_CHANNELS

NUM_CORES = 2
NUM_SUBCORES = 16
HALF = TOTAL // NUM_CORES
SHARD = HALF // NUM_SUBCORES
ZCHUNK = 32768
NZDMA = SHARD // ZCHUNK

ICOLS = 128
REL_PER_TILE = N_REL // NUM_SUBCORES
IROWS = REL_PER_TILE // ICOLS
VSTEPS = REL_PER_TILE // 16
IGNORED = -1
DMA_GROUP = 8

_mesh = plsc.VectorSubcoreMesh(
    core_axis_name="c", subcore_axis_name="s",
    num_cores=NUM_CORES, num_subcores=NUM_SUBCORES,
)


@functools.partial(
    pl.kernel,
    mesh=_mesh,
    out_type=jax.ShapeDtypeStruct((TOTAL,), jnp.float32),
    scratch_types=[
        pltpu.VMEM((ZCHUNK,), jnp.float32),
        pltpu.VMEM((REL_PER_TILE,), jnp.int32),
        pltpu.VMEM((REL_PER_TILE,), jnp.int32),
        pltpu.VMEM((REL_PER_TILE,), jnp.int32),
        pltpu.VMEM((REL_PER_TILE,), jnp.int32),
        pltpu.VMEM((REL_PER_TILE,), jnp.float32),
        pltpu.VMEM((IROWS, ICOLS), jnp.int32),
        pltpu.SemaphoreType.DMA,
        pltpu.SemaphoreType.DMA,
    ],
)
def _sc_dep(rb_hbm, rs_hbm, rd_hbm, rt_hbm, rv_hbm, dep_hbm,
            zero_v, b_v, s_v, d_v, t_v, v_v, idx_v, zsem, sem):
    cid = lax.axis_index("c")
    sid = lax.axis_index("s")

    def fill_zero(i, carry):
        zero_v[pl.ds(i * 16, 16)] = jnp.zeros((16,), jnp.float32)
        return carry

    lax.fori_loop(0, ZCHUNK // 16, fill_zero, 0)

    shard_base = cid * HALF + sid * SHARD
    zero_copies = [
        pltpu.make_async_copy(
            zero_v, dep_hbm.at[pl.ds(shard_base + k * ZCHUNK, ZCHUNK)], zsem
        )
        for k in range(NZDMA)
    ]
    for cp in zero_copies:
        cp.start()

    rel_base = sid * REL_PER_TILE
    in_copies = [
        pltpu.async_copy(hbm.at[pl.ds(rel_base, REL_PER_TILE)], vmem, sem)
        for hbm, vmem in [(rb_hbm, b_v), (rs_hbm, s_v), (rd_hbm, d_v),
                          (rt_hbm, t_v), (rv_hbm, v_v)]
    ]
    for cp in in_copies:
        cp.wait()

    lo = cid * HALF
    hi = lo + HALF

    def compute(m, carry):
        p = m * 16
        b = b_v[pl.ds(p, 16)]
        s = s_v[pl.ds(p, 16)]
        d = d_v[pl.ds(p, 16)]
        t = t_v[pl.ds(p, 16)]
        off = (
            (b * N_CHANNELS + t) * (LENGTH * LENGTH)
            + (((s >> 3) << 2) + (d >> 7)) * 1024
            + ((s & 7) << 7)
            + (d & 127)
        )
        owned = (off >= lo) & (off < hi)
        r = m // 8
        c16 = (m % 8) * 16
        idx_v[r, pl.ds(c16, 16)] = jnp.where(
            owned, off, jnp.full((16,), IGNORED, jnp.int32)
        )
        return carry

    lax.fori_loop(0, VSTEPS, compute, 0)

    for cp in zero_copies:
        cp.wait()
    plsc.subcore_barrier()

    def group(g, carry):
        row0 = g * DMA_GROUP
        copies = [
            pltpu.async_copy(
                v_v.at[pl.ds((row0 + j) * ICOLS, ICOLS)],
                dep_hbm.at[plsc.Indices(idx_v.at[row0 + j],
                                        ignored_value=IGNORED)],
                sem,
            )
            for j in range(DMA_GROUP)
        ]
        for cp in copies:
            cp.wait()
        return carry

    lax.fori_loop(0, IROWS // DMA_GROUP, group, 0)


def kernel(rel_b, rel_src, rel_dst, rel_type, rel_val):
    dep = _sc_dep(rel_b, rel_src, rel_dst, rel_type, rel_val)
    x = dep.reshape(BATCH, N_CHANNELS, LENGTH // 8, LENGTH // 128, 8, 128)
    x = x.transpose(0, 2, 4, 3, 5, 1)
    return x.reshape(BATCH, LENGTH, LENGTH, N_CHANNELS)
